# baseline (speedup 1.0000x reference)
import functools

import jax
import jax.numpy as jnp
from jax import lax
from jax.experimental import pallas as pl
from jax.experimental.pallas import tpu as pltpu
from jax.experimental.pallas import tpu_sc as plsc

_N = 10000
_E = 320000
_D = 128

_NW = 32
_CH = 128
_NCH = _E // _CH
_PERW = -(-_NCH // _NW)

_F32 = jnp.float32


def _dotT(x, w):
    return lax.dot_general(x, w, (((1,), (1,)), ((), ())),
                           preferred_element_type=_F32)


def _pre_body(h_ref, wr_ref, wc_ref, be1_ref, a_ref, b_ref):
    h = h_ref[...]
    a_ref[...] = _dotT(h, wr_ref[...]) + be1_ref[...][None, :]
    b_ref[...] = _dotT(h, wc_ref[...])


_pre = pl.pallas_call(
    _pre_body,
    grid=(10,),
    in_specs=[
        pl.BlockSpec((1000, _D), lambda i: (i, 0)),
        pl.BlockSpec((_D, _D), lambda i: (0, 0)),
        pl.BlockSpec((_D, _D), lambda i: (0, 0)),
        pl.BlockSpec((_D,), lambda i: (0,)),
    ],
    out_specs=[pl.BlockSpec((1000, _D), lambda i: (i, 0))] * 2,
    out_shape=[jax.ShapeDtypeStruct((_N, _D), _F32)] * 2,
)


def _make_edge_gather(ne):
  nch = ne // _CH
  perw = -(-nch // _NW)

  def _edge_gather_body(a_hbm, b_hbm, cx_hbm, cy_hbm, cz_hbm, row_hbm, col_hbm,
                      w1_hbm,
                      u_hbm, rx_hbm, ry_hbm, rz_hbm,
                      idxr, idxc, bufa, bufb, crx, cry, crz, ccx, ccy, ccz,
                      brx, bry, brz, sqbuf0, sqbuf1, w1vm,
                      semi0, semi1, semg0, semg1, semw0, semw1):
      cid = lax.axis_index("c")
      sid = lax.axis_index("s")
      wid = sid * 2 + cid
      pltpu.sync_copy(w1_hbm, w1vm)
      w1c = [w1vm[pl.ds(i * 16, 16)] for i in range(8)]
      nloc = jnp.minimum(perw, nch - wid * perw)
      semi = (semi0, semi1)
      semg = (semg0, semg1)
      semw = (semw0, semw1)

      def base_of(k):
          return (wid * perw + k) * _CH

      def fire_idx(k, p):
          b = base_of(k)
          pltpu.async_copy(row_hbm.at[pl.ds(b, _CH)], idxr.at[p], semi[p])
          pltpu.async_copy(col_hbm.at[pl.ds(b, _CH)], idxc.at[p], semi[p])

      def wait_idx(p):
          pltpu.make_async_copy(row_hbm.at[pl.ds(0, _CH)], idxr.at[p],
                                semi[p]).wait()
          pltpu.make_async_copy(col_hbm.at[pl.ds(0, _CH)], idxc.at[p],
                                semi[p]).wait()

      def gather_copies(p):
          s = semg[p]
          return (
              pltpu.make_async_copy(a_hbm.at[idxr.at[p]], bufa.at[p], s),
              pltpu.make_async_copy(b_hbm.at[idxc.at[p]], bufb.at[p], s),
              pltpu.make_async_copy(cx_hbm.at[idxr.at[p]], crx.at[p], s),
              pltpu.make_async_copy(cy_hbm.at[idxr.at[p]], cry.at[p], s),
              pltpu.make_async_copy(cz_hbm.at[idxr.at[p]], crz.at[p], s),
              pltpu.make_async_copy(cx_hbm.at[idxc.at[p]], ccx.at[p], s),
              pltpu.make_async_copy(cy_hbm.at[idxc.at[p]], ccy.at[p], s),
              pltpu.make_async_copy(cz_hbm.at[idxc.at[p]], ccz.at[p], s),
          )

      def fire_gather(p):
          for c in gather_copies(p):
              c.start()

      def wait_gather(p):
          for c in gather_copies(p):
              c.wait()

      def write_copies(k, p):
          b = base_of(k)
          s = semw[p]
          return (
              pltpu.make_async_copy(bufa.at[p], u_hbm.at[pl.ds(b, _CH)], s),
              pltpu.make_async_copy(brx.at[p], rx_hbm.at[pl.ds(b, _CH)], s),
              pltpu.make_async_copy(bry.at[p], ry_hbm.at[pl.ds(b, _CH)], s),
              pltpu.make_async_copy(brz.at[p], rz_hbm.at[pl.ds(b, _CH)], s),
          )

      def fire_writes(k, p):
          for c in write_copies(k, p):
              c.start()

      def wait_writes(p):
          for c in write_copies(0, p):
              c.wait()

      def compute(p):
          sq_b = (sqbuf0, sqbuf1)[p]
          for g in range(_CH // 16):
              sl = pl.ds(g * 16, 16)
              dx = crx[p, sl] - ccx[p, sl]
              dy = cry[p, sl] - ccy[p, sl]
              dz = crz[p, sl] - ccz[p, sl]
              brx[p, sl] = dx
              bry[p, sl] = dy
              brz[p, sl] = dz
              sq_b[sl] = dx * dx + dy * dy + dz * dz

          @pl.loop(0, _CH, unroll=4)
          def _edge(e):
              s = sq_b[pl.ds(e, 16)][0]
              for i in range(8):
                  sl = pl.ds(i * 16, 16)
                  z = bufa[p, e, sl] + bufb[p, e, sl] + s * w1c[i]
                  bufa[p, e, sl] = jnp.maximum(z, 0.0)

      fire_idx(0, 0)
      fire_idx(1, 1)
      wait_idx(0)
      fire_gather(0)

      @pl.loop(0, (nloc + 1) // 2)
      def _t(t):
          for p in (0, 1):
              k = 2 * t + p

              @pl.when(k < nloc)
              def _():
                  wait_gather(p)

                  @pl.when(k + 2 < nloc)
                  def _():
                      fire_idx(k + 2, p)

                  @pl.when(k + 1 < nloc)
                  def _():
                      wait_idx(1 - p)

                      @pl.when(k >= 1)
                      def _():
                          wait_writes(1 - p)

                      fire_gather(1 - p)

                  compute(p)
                  fire_writes(k, p)

      wait_writes(0)
      wait_writes(1)


  return pl.kernel(
    _edge_gather_body,
    out_type=(jax.ShapeDtypeStruct((ne, _D), _F32),
              jax.ShapeDtypeStruct((ne,), _F32),
              jax.ShapeDtypeStruct((ne,), _F32),
              jax.ShapeDtypeStruct((ne,), _F32)),
    mesh=plsc.VectorSubcoreMesh(core_axis_name="c", subcore_axis_name="s",
                                num_cores=2, num_subcores=16),
    scratch_types=(
        pltpu.VMEM((2, _CH), jnp.int32),
        pltpu.VMEM((2, _CH), jnp.int32),
        pltpu.VMEM((2, _CH, _D), _F32),
        pltpu.VMEM((2, _CH, _D), _F32),
        pltpu.VMEM((2, _CH), _F32),
        pltpu.VMEM((2, _CH), _F32),
        pltpu.VMEM((2, _CH), _F32),
        pltpu.VMEM((2, _CH), _F32),
        pltpu.VMEM((2, _CH), _F32),
        pltpu.VMEM((2, _CH), _F32),
        pltpu.VMEM((2, _CH), _F32),
        pltpu.VMEM((2, _CH), _F32),
        pltpu.VMEM((2, _CH), _F32),
        pltpu.VMEM((_CH + 16,), _F32),
        pltpu.VMEM((_CH + 16,), _F32),
        pltpu.VMEM((_D,), _F32),
        pltpu.SemaphoreType.DMA,
        pltpu.SemaphoreType.DMA,
        pltpu.SemaphoreType.DMA,
        pltpu.SemaphoreType.DMA,
        pltpu.SemaphoreType.DMA,
        pltpu.SemaphoreType.DMA,
    ),
  )


_edge_gather_f = _make_edge_gather(_E)


def _emlp_body(u_ref, we2_ref, be2_ref, wc1_ref, bc1_ref, wc2_ref,
               m_ref, cw_ref):
    u = u_ref[...]
    m = _dotT(u, we2_ref[...]) + be2_ref[...][None, :]
    m_ref[...] = m
    t = jnp.maximum(_dotT(m, wc1_ref[...]) + bc1_ref[...][None, :], 0.0)
    cw_ref[...] = jnp.sum(t * wc2_ref[...][None, :], axis=1)


_EBLK = 512
_emlp_f = pl.pallas_call(
    _emlp_body,
    grid=(_E // _EBLK,),
    in_specs=[
        pl.BlockSpec((_EBLK, _D), lambda i: (i, 0)),
        pl.BlockSpec((_D, _D), lambda i: (0, 0)),
        pl.BlockSpec((_D,), lambda i: (0,)),
        pl.BlockSpec((_D, _D), lambda i: (0, 0)),
        pl.BlockSpec((_D,), lambda i: (0,)),
        pl.BlockSpec((_D,), lambda i: (0,)),
    ],
    out_specs=[
        pl.BlockSpec((_EBLK, _D), lambda i: (i, 0)),
        pl.BlockSpec((_EBLK,), lambda i: (i,)),
    ],
    out_shape=[
        jax.ShapeDtypeStruct((_E, _D), _F32),
        jax.ShapeDtypeStruct((_E,), _F32),
    ],
)


_NPT = 1000


def _make_scatter(ne):
  nch = ne // _CH
  perw = -(-nch // _NW)

  def _scatter_body(m_hbm, cw_hbm, rx_hbm, ry_hbm, rz_hbm, row_hbm,
                  aggp_hbm, cux0_hbm, cux1_hbm, cuy0_hbm, cuy1_hbm,
                  cuz0_hbm, cuz1_hbm,
                  idx, mbuf, cwbuf, brx, bry, brz, wx, wy, wz, ob1, zbm,
                  accm, accx, accy, accz,
                  seml0, seml1, sems0, sems1):
      cid = lax.axis_index("c")
      sid = lax.axis_index("s")
      wid = sid * 2 + cid

      @pl.loop(0, 40)
      def _zm(r):
          for i in range(8):
              zbm[r, pl.ds(i * 16, 16)] = jnp.zeros((16,), _F32)

      @pl.loop(0, 63)
      def _zo(r):
          ob1[pl.ds(r * 16, 16)] = jnp.zeros((16,), _F32)

      @pl.when(sid < 10)
      def _zero():
          for c in range(25):
              pltpu.sync_copy(
                  zbm,
                  accm.at[pl.ds(sid * _NPT + c * 40, 40)])
          sl = pl.ds(sid * _NPT, _NPT)
          pltpu.sync_copy(ob1.at[pl.ds(0, _NPT)], accx.at[sl])
          pltpu.sync_copy(ob1.at[pl.ds(0, _NPT)], accy.at[sl])
          pltpu.sync_copy(ob1.at[pl.ds(0, _NPT)], accz.at[sl])

      plsc.subcore_barrier()

      nloc = jnp.minimum(perw, nch - wid * perw)
      seml = (seml0, seml1)
      sems = (sems0, sems1)

      def base_of(k):
          return (wid * perw + k) * _CH

      def load_copies(k, p):
          b = base_of(k)
          s = seml[p]
          return (
              pltpu.make_async_copy(row_hbm.at[pl.ds(b, _CH)], idx.at[p], s),
              pltpu.make_async_copy(m_hbm.at[pl.ds(b, _CH)], mbuf.at[p], s),
              pltpu.make_async_copy(cw_hbm.at[pl.ds(b, _CH)], cwbuf.at[p], s),
              pltpu.make_async_copy(rx_hbm.at[pl.ds(b, _CH)], brx.at[p], s),
              pltpu.make_async_copy(ry_hbm.at[pl.ds(b, _CH)], bry.at[p], s),
              pltpu.make_async_copy(rz_hbm.at[pl.ds(b, _CH)], brz.at[p], s),
          )

      def fire_loads(k, p):
          for c in load_copies(k, p):
              c.start()

      def wait_loads(p):
          for c in load_copies(0, p):
              c.wait()

      def scat_copies(p):
          s = sems[p]
          return (
              pltpu.make_async_copy(mbuf.at[p], accm.at[idx.at[p]], s),
              pltpu.make_async_copy(wx.at[p], accx.at[idx.at[p]], s),
              pltpu.make_async_copy(wy.at[p], accy.at[idx.at[p]], s),
              pltpu.make_async_copy(wz.at[p], accz.at[idx.at[p]], s),
          )

      def fire_scat(p):
          for c in scat_copies(p):
              c.start(add=True)

      def wait_scat(p):
          for c in scat_copies(p):
              c.wait()

      fire_loads(0, 0)

      @pl.loop(0, (nloc + 1) // 2)
      def _t(t):
          for p in (0, 1):
              k = 2 * t + p

              @pl.when(k < nloc)
              def _():
                  wait_loads(p)

                  @pl.when(k + 1 < nloc)
                  def _():
                      @pl.when(k >= 1)
                      def _():
                          wait_scat(1 - p)

                      fire_loads(k + 1, 1 - p)

                  for g in range(_CH // 16):
                      sl = pl.ds(g * 16, 16)
                      cw16 = cwbuf[p, sl]
                      wx[p, sl] = cw16 * brx[p, sl]
                      wy[p, sl] = cw16 * bry[p, sl]
                      wz[p, sl] = cw16 * brz[p, sl]
                  fire_scat(p)

      wait_scat(0)
      wait_scat(1)

      plsc.subcore_barrier()

      @pl.when(sid < 10)
      def _out():
          for c in range(25):
              pltpu.sync_copy(
                  accm.at[pl.ds(sid * _NPT + c * 40, 40)],
                  zbm)
              pltpu.sync_copy(
                  zbm,
                  aggp_hbm.at[pl.ds(cid * _N + sid * _NPT + c * 40, 40)])
          sl = pl.ds(sid * _NPT, _NPT)
          for acc, o0_hbm, o1_hbm in ((accx, cux0_hbm, cux1_hbm),
                                      (accy, cuy0_hbm, cuy1_hbm),
                                      (accz, cuz0_hbm, cuz1_hbm)):
              pltpu.sync_copy(acc.at[sl], ob1.at[pl.ds(0, _NPT)])

              @pl.when(cid == 0)
              def _o0():
                  pltpu.sync_copy(ob1.at[pl.ds(0, _NPT)], o0_hbm.at[sl])

              @pl.when(cid == 1)
              def _o1():
                  pltpu.sync_copy(ob1.at[pl.ds(0, _NPT)], o1_hbm.at[sl])


  return pl.kernel(
    _scatter_body,
    out_type=(jax.ShapeDtypeStruct((2 * _N, _D), _F32),)
             + (jax.ShapeDtypeStruct((_N,), _F32),) * 6,
    mesh=plsc.VectorSubcoreMesh(core_axis_name="c", subcore_axis_name="s",
                                num_cores=2, num_subcores=16),
    scratch_types=(
        pltpu.VMEM((2, _CH), jnp.int32),
        pltpu.VMEM((2, _CH, _D), _F32),
        pltpu.VMEM((2, _CH), _F32),
        pltpu.VMEM((2, _CH), _F32),
        pltpu.VMEM((2, _CH), _F32),
        pltpu.VMEM((2, _CH), _F32),
        pltpu.VMEM((2, _CH), _F32),
        pltpu.VMEM((2, _CH), _F32),
        pltpu.VMEM((2, _CH), _F32),
        pltpu.VMEM((1008,), _F32),
        pltpu.VMEM((40, _D), _F32),
        pltpu.VMEM_SHARED((_N, _D), _F32),
        pltpu.VMEM_SHARED((_N,), _F32),
        pltpu.VMEM_SHARED((_N,), _F32),
        pltpu.VMEM_SHARED((_N,), _F32),
        pltpu.SemaphoreType.DMA,
        pltpu.SemaphoreType.DMA,
        pltpu.SemaphoreType.DMA,
        pltpu.SemaphoreType.DMA,
    ),
  )


_scatter_f = _make_scatter(_E)


def _nmlp_body(h_ref, g0_ref, g1_ref, wh_ref, wg_ref,
               bn1_ref, wn2_ref, bn2_ref,
               cx_ref, cy_ref, cz_ref, x0, x1, y0, y1, z0, z1,
               hout_ref, cox_ref, coy_ref, coz_ref):
    @pl.when(pl.program_id(0) == 0)
    def _coords():
        cox_ref[...] = cx_ref[...] + x0[...] + x1[...]
        coy_ref[...] = cy_ref[...] + y0[...] + y1[...]
        coz_ref[...] = cz_ref[...] + z0[...] + z1[...]

    g = g0_ref[...] + g1_ref[...]
    pre = (_dotT(h_ref[...], wh_ref[...]) + _dotT(g, wg_ref[...])
           + bn1_ref[...][None, :])
    hm = jnp.maximum(pre, 0.0)
    hout_ref[...] = _dotT(hm, wn2_ref[...]) + bn2_ref[...][None, :]


_NBLK = 1000
_nmlp = pl.pallas_call(
    _nmlp_body,
    grid=(_N // _NBLK,),
    in_specs=[
        pl.BlockSpec((_NBLK, _D), lambda i: (i, 0)),
        pl.BlockSpec((_NBLK, _D), lambda i: (i, 0)),
        pl.BlockSpec((_NBLK, _D), lambda i: (i + _N // _NBLK, 0)),
        pl.BlockSpec((_D, _D), lambda i: (0, 0)),
        pl.BlockSpec((_D, _D), lambda i: (0, 0)),
        pl.BlockSpec((_D,), lambda i: (0,)),
        pl.BlockSpec((_D, _D), lambda i: (0, 0)),
        pl.BlockSpec((_D,), lambda i: (0,)),
    ] + [pl.BlockSpec((_N,), lambda i: (0,))] * 9,
    out_specs=[pl.BlockSpec((_NBLK, _D), lambda i: (i, 0))]
    + [pl.BlockSpec((_N,), lambda i: (0,))] * 3,
    out_shape=[jax.ShapeDtypeStruct((_N, _D), _F32)]
    + [jax.ShapeDtypeStruct((_N,), _F32)] * 3,
)


@jax.jit
def kernel(h, coords, edge_index, We1, be1, We2, be2, Wn1, bn1, Wn2, bn2,
           Wc1, bc1, Wc2):
    row = edge_index[0]
    col = edge_index[1]
    wr = We1[:, :_D]
    wcl = We1[:, _D:2 * _D]
    w1d = We1[:, 2 * _D]
    cx = coords[:, 0]
    cy = coords[:, 1]
    cz = coords[:, 2]

    a, b = _pre(h, wr, wcl, be1)
    u, rx, ry, rz = _edge_gather_f(a, b, cx, cy, cz, row, col, w1d)
    m, cw = _emlp_f(u, We2, be2, Wc1, bc1, Wc2[0])
    (aggp, cux0, cux1, cuy0, cuy1, cuz0, cuz1) = _scatter_f(
        m, cw, rx, ry, rz, row)
    wh = Wn1[:, :_D]
    wg = Wn1[:, _D:]
    h_new, cox, coy, coz = _nmlp(h, aggp, aggp, wh, wg, bn1, Wn2, bn2,
                                 cx, cy, cz, cux0, cux1, cuy0, cuy1,
                                 cuz0, cuz1)
    return h_new, jnp.stack([cox, coy, coz], axis=1)


# speedup vs baseline: 1.0004x; 1.0004x over previous
import functools

import jax
import jax.numpy as jnp
from jax import lax
from jax.experimental import pallas as pl
from jax.experimental.pallas import tpu as pltpu
from jax.experimental.pallas import tpu_sc as plsc

_N = 10000
_E = 320000
_D = 128

_NW = 32
_CH = 128
_NCH = _E // _CH
_PERW = -(-_NCH // _NW)

_F32 = jnp.float32


def _dotT(x, w):
    return lax.dot_general(x, w, (((1,), (1,)), ((), ())),
                           preferred_element_type=_F32)


def _pre_body(h_ref, wr_ref, wc_ref, be1_ref, a_ref, b_ref):
    h = h_ref[...]
    a_ref[...] = _dotT(h, wr_ref[...]) + be1_ref[...][None, :]
    b_ref[...] = _dotT(h, wc_ref[...])


_pre = pl.pallas_call(
    _pre_body,
    grid=(10,),
    in_specs=[
        pl.BlockSpec((1000, _D), lambda i: (i, 0)),
        pl.BlockSpec((_D, _D), lambda i: (0, 0)),
        pl.BlockSpec((_D, _D), lambda i: (0, 0)),
        pl.BlockSpec((_D,), lambda i: (0,)),
    ],
    out_specs=[pl.BlockSpec((1000, _D), lambda i: (i, 0))] * 2,
    out_shape=[jax.ShapeDtypeStruct((_N, _D), _F32)] * 2,
)


def _make_edge_gather(ne):
  nch = ne // _CH
  perw = -(-nch // _NW)

  def _edge_gather_body(a_hbm, b_hbm, cx_hbm, cy_hbm, cz_hbm, row_hbm, col_hbm,
                      w1_hbm,
                      u_hbm, rx_hbm, ry_hbm, rz_hbm,
                      idxr, idxc, bufa, bufb, crx, cry, crz, ccx, ccy, ccz,
                      brx, bry, brz, sqbuf0, sqbuf1, w1vm,
                      semi0, semi1, semg0, semg1, semw0, semw1):
      cid = lax.axis_index("c")
      sid = lax.axis_index("s")
      wid = sid * 2 + cid
      pltpu.sync_copy(w1_hbm, w1vm)
      w1c = [w1vm[pl.ds(i * 16, 16)] for i in range(8)]
      nloc = jnp.minimum(perw, nch - wid * perw)
      semi = (semi0, semi1)
      semg = (semg0, semg1)
      semw = (semw0, semw1)

      def base_of(k):
          return (wid * perw + k) * _CH

      def fire_idx(k, p):
          b = base_of(k)
          pltpu.async_copy(row_hbm.at[pl.ds(b, _CH)], idxr.at[p], semi[p])
          pltpu.async_copy(col_hbm.at[pl.ds(b, _CH)], idxc.at[p], semi[p])

      def wait_idx(p):
          pltpu.make_async_copy(row_hbm.at[pl.ds(0, _CH)], idxr.at[p],
                                semi[p]).wait()
          pltpu.make_async_copy(col_hbm.at[pl.ds(0, _CH)], idxc.at[p],
                                semi[p]).wait()

      def gather_copies(p):
          s = semg[p]
          return (
              pltpu.make_async_copy(a_hbm.at[idxr.at[p]], bufa.at[p], s),
              pltpu.make_async_copy(b_hbm.at[idxc.at[p]], bufb.at[p], s),
              pltpu.make_async_copy(cx_hbm.at[idxr.at[p]], crx.at[p], s),
              pltpu.make_async_copy(cy_hbm.at[idxr.at[p]], cry.at[p], s),
              pltpu.make_async_copy(cz_hbm.at[idxr.at[p]], crz.at[p], s),
              pltpu.make_async_copy(cx_hbm.at[idxc.at[p]], ccx.at[p], s),
              pltpu.make_async_copy(cy_hbm.at[idxc.at[p]], ccy.at[p], s),
              pltpu.make_async_copy(cz_hbm.at[idxc.at[p]], ccz.at[p], s),
          )

      def fire_gather(p):
          for c in gather_copies(p):
              c.start()

      def wait_gather(p):
          for c in gather_copies(p):
              c.wait()

      def write_copies(k, p):
          b = base_of(k)
          s = semw[p]
          return (
              pltpu.make_async_copy(bufa.at[p], u_hbm.at[pl.ds(b, _CH)], s),
              pltpu.make_async_copy(brx.at[p], rx_hbm.at[pl.ds(b, _CH)], s),
              pltpu.make_async_copy(bry.at[p], ry_hbm.at[pl.ds(b, _CH)], s),
              pltpu.make_async_copy(brz.at[p], rz_hbm.at[pl.ds(b, _CH)], s),
          )

      def fire_writes(k, p):
          for c in write_copies(k, p):
              c.start()

      def wait_writes(p):
          for c in write_copies(0, p):
              c.wait()

      def compute(p):
          sq_b = (sqbuf0, sqbuf1)[p]
          for g in range(_CH // 16):
              sl = pl.ds(g * 16, 16)
              dx = crx[p, sl] - ccx[p, sl]
              dy = cry[p, sl] - ccy[p, sl]
              dz = crz[p, sl] - ccz[p, sl]
              brx[p, sl] = dx
              bry[p, sl] = dy
              brz[p, sl] = dz
              sq_b[sl] = dx * dx + dy * dy + dz * dz

          @pl.loop(0, _CH, unroll=2)
          def _edge(e):
              s = sq_b[pl.ds(e, 16)][0]
              for i in range(8):
                  sl = pl.ds(i * 16, 16)
                  z = bufa[p, e, sl] + bufb[p, e, sl] + s * w1c[i]
                  bufa[p, e, sl] = jnp.maximum(z, 0.0)

      fire_idx(0, 0)
      fire_idx(1, 1)
      wait_idx(0)
      fire_gather(0)

      @pl.loop(0, (nloc + 1) // 2)
      def _t(t):
          for p in (0, 1):
              k = 2 * t + p

              @pl.when(k < nloc)
              def _():
                  wait_gather(p)

                  @pl.when(k + 2 < nloc)
                  def _():
                      fire_idx(k + 2, p)

                  @pl.when(k + 1 < nloc)
                  def _():
                      wait_idx(1 - p)

                      @pl.when(k >= 1)
                      def _():
                          wait_writes(1 - p)

                      fire_gather(1 - p)

                  compute(p)
                  fire_writes(k, p)

      wait_writes(0)
      wait_writes(1)


  return pl.kernel(
    _edge_gather_body,
    out_type=(jax.ShapeDtypeStruct((ne, _D), _F32),
              jax.ShapeDtypeStruct((ne,), _F32),
              jax.ShapeDtypeStruct((ne,), _F32),
              jax.ShapeDtypeStruct((ne,), _F32)),
    mesh=plsc.VectorSubcoreMesh(core_axis_name="c", subcore_axis_name="s",
                                num_cores=2, num_subcores=16),
    scratch_types=(
        pltpu.VMEM((2, _CH), jnp.int32),
        pltpu.VMEM((2, _CH), jnp.int32),
        pltpu.VMEM((2, _CH, _D), _F32),
        pltpu.VMEM((2, _CH, _D), _F32),
        pltpu.VMEM((2, _CH), _F32),
        pltpu.VMEM((2, _CH), _F32),
        pltpu.VMEM((2, _CH), _F32),
        pltpu.VMEM((2, _CH), _F32),
        pltpu.VMEM((2, _CH), _F32),
        pltpu.VMEM((2, _CH), _F32),
        pltpu.VMEM((2, _CH), _F32),
        pltpu.VMEM((2, _CH), _F32),
        pltpu.VMEM((2, _CH), _F32),
        pltpu.VMEM((_CH + 16,), _F32),
        pltpu.VMEM((_CH + 16,), _F32),
        pltpu.VMEM((_D,), _F32),
        pltpu.SemaphoreType.DMA,
        pltpu.SemaphoreType.DMA,
        pltpu.SemaphoreType.DMA,
        pltpu.SemaphoreType.DMA,
        pltpu.SemaphoreType.DMA,
        pltpu.SemaphoreType.DMA,
    ),
  )


_edge_gather_f = _make_edge_gather(_E)


def _emlp_body(u_ref, we2_ref, be2_ref, wc1_ref, bc1_ref, wc2_ref,
               m_ref, cw_ref):
    u = u_ref[...]
    m = _dotT(u, we2_ref[...]) + be2_ref[...][None, :]
    m_ref[...] = m
    t = jnp.maximum(_dotT(m, wc1_ref[...]) + bc1_ref[...][None, :], 0.0)
    cw_ref[...] = jnp.sum(t * wc2_ref[...][None, :], axis=1)


_EBLK = 512
_emlp_f = pl.pallas_call(
    _emlp_body,
    grid=(_E // _EBLK,),
    in_specs=[
        pl.BlockSpec((_EBLK, _D), lambda i: (i, 0)),
        pl.BlockSpec((_D, _D), lambda i: (0, 0)),
        pl.BlockSpec((_D,), lambda i: (0,)),
        pl.BlockSpec((_D, _D), lambda i: (0, 0)),
        pl.BlockSpec((_D,), lambda i: (0,)),
        pl.BlockSpec((_D,), lambda i: (0,)),
    ],
    out_specs=[
        pl.BlockSpec((_EBLK, _D), lambda i: (i, 0)),
        pl.BlockSpec((_EBLK,), lambda i: (i,)),
    ],
    out_shape=[
        jax.ShapeDtypeStruct((_E, _D), _F32),
        jax.ShapeDtypeStruct((_E,), _F32),
    ],
)


_NPT = 1000


def _make_scatter(ne):
  nch = ne // _CH
  perw = -(-nch // _NW)

  def _scatter_body(m_hbm, cw_hbm, rx_hbm, ry_hbm, rz_hbm, row_hbm,
                  aggp_hbm, cux0_hbm, cux1_hbm, cuy0_hbm, cuy1_hbm,
                  cuz0_hbm, cuz1_hbm,
                  idx, mbuf, cwbuf, brx, bry, brz, wx, wy, wz, ob1, zbm,
                  accm, accx, accy, accz,
                  seml0, seml1, sems0, sems1):
      cid = lax.axis_index("c")
      sid = lax.axis_index("s")
      wid = sid * 2 + cid

      @pl.loop(0, 40)
      def _zm(r):
          for i in range(8):
              zbm[r, pl.ds(i * 16, 16)] = jnp.zeros((16,), _F32)

      @pl.loop(0, 63)
      def _zo(r):
          ob1[pl.ds(r * 16, 16)] = jnp.zeros((16,), _F32)

      @pl.when(sid < 10)
      def _zero():
          for c in range(25):
              pltpu.sync_copy(
                  zbm,
                  accm.at[pl.ds(sid * _NPT + c * 40, 40)])
          sl = pl.ds(sid * _NPT, _NPT)
          pltpu.sync_copy(ob1.at[pl.ds(0, _NPT)], accx.at[sl])
          pltpu.sync_copy(ob1.at[pl.ds(0, _NPT)], accy.at[sl])
          pltpu.sync_copy(ob1.at[pl.ds(0, _NPT)], accz.at[sl])

      plsc.subcore_barrier()

      nloc = jnp.minimum(perw, nch - wid * perw)
      seml = (seml0, seml1)
      sems = (sems0, sems1)

      def base_of(k):
          return (wid * perw + k) * _CH

      def load_copies(k, p):
          b = base_of(k)
          s = seml[p]
          return (
              pltpu.make_async_copy(row_hbm.at[pl.ds(b, _CH)], idx.at[p], s),
              pltpu.make_async_copy(m_hbm.at[pl.ds(b, _CH)], mbuf.at[p], s),
              pltpu.make_async_copy(cw_hbm.at[pl.ds(b, _CH)], cwbuf.at[p], s),
              pltpu.make_async_copy(rx_hbm.at[pl.ds(b, _CH)], brx.at[p], s),
              pltpu.make_async_copy(ry_hbm.at[pl.ds(b, _CH)], bry.at[p], s),
              pltpu.make_async_copy(rz_hbm.at[pl.ds(b, _CH)], brz.at[p], s),
          )

      def fire_loads(k, p):
          for c in load_copies(k, p):
              c.start()

      def wait_loads(p):
          for c in load_copies(0, p):
              c.wait()

      def scat_copies(p):
          s = sems[p]
          return (
              pltpu.make_async_copy(mbuf.at[p], accm.at[idx.at[p]], s),
              pltpu.make_async_copy(wx.at[p], accx.at[idx.at[p]], s),
              pltpu.make_async_copy(wy.at[p], accy.at[idx.at[p]], s),
              pltpu.make_async_copy(wz.at[p], accz.at[idx.at[p]], s),
          )

      def fire_scat(p):
          for c in scat_copies(p):
              c.start(add=True)

      def wait_scat(p):
          for c in scat_copies(p):
              c.wait()

      fire_loads(0, 0)

      @pl.loop(0, (nloc + 1) // 2)
      def _t(t):
          for p in (0, 1):
              k = 2 * t + p

              @pl.when(k < nloc)
              def _():
                  wait_loads(p)

                  @pl.when(k + 1 < nloc)
                  def _():
                      @pl.when(k >= 1)
                      def _():
                          wait_scat(1 - p)

                      fire_loads(k + 1, 1 - p)

                  for g in range(_CH // 16):
                      sl = pl.ds(g * 16, 16)
                      cw16 = cwbuf[p, sl]
                      wx[p, sl] = cw16 * brx[p, sl]
                      wy[p, sl] = cw16 * bry[p, sl]
                      wz[p, sl] = cw16 * brz[p, sl]
                  fire_scat(p)

      wait_scat(0)
      wait_scat(1)

      plsc.subcore_barrier()

      @pl.when(sid < 10)
      def _out():
          for c in range(25):
              pltpu.sync_copy(
                  accm.at[pl.ds(sid * _NPT + c * 40, 40)],
                  zbm)
              pltpu.sync_copy(
                  zbm,
                  aggp_hbm.at[pl.ds(cid * _N + sid * _NPT + c * 40, 40)])
          sl = pl.ds(sid * _NPT, _NPT)
          for acc, o0_hbm, o1_hbm in ((accx, cux0_hbm, cux1_hbm),
                                      (accy, cuy0_hbm, cuy1_hbm),
                                      (accz, cuz0_hbm, cuz1_hbm)):
              pltpu.sync_copy(acc.at[sl], ob1.at[pl.ds(0, _NPT)])

              @pl.when(cid == 0)
              def _o0():
                  pltpu.sync_copy(ob1.at[pl.ds(0, _NPT)], o0_hbm.at[sl])

              @pl.when(cid == 1)
              def _o1():
                  pltpu.sync_copy(ob1.at[pl.ds(0, _NPT)], o1_hbm.at[sl])


  return pl.kernel(
    _scatter_body,
    out_type=(jax.ShapeDtypeStruct((2 * _N, _D), _F32),)
             + (jax.ShapeDtypeStruct((_N,), _F32),) * 6,
    mesh=plsc.VectorSubcoreMesh(core_axis_name="c", subcore_axis_name="s",
                                num_cores=2, num_subcores=16),
    scratch_types=(
        pltpu.VMEM((2, _CH), jnp.int32),
        pltpu.VMEM((2, _CH, _D), _F32),
        pltpu.VMEM((2, _CH), _F32),
        pltpu.VMEM((2, _CH), _F32),
        pltpu.VMEM((2, _CH), _F32),
        pltpu.VMEM((2, _CH), _F32),
        pltpu.VMEM((2, _CH), _F32),
        pltpu.VMEM((2, _CH), _F32),
        pltpu.VMEM((2, _CH), _F32),
        pltpu.VMEM((1008,), _F32),
        pltpu.VMEM((40, _D), _F32),
        pltpu.VMEM_SHARED((_N, _D), _F32),
        pltpu.VMEM_SHARED((_N,), _F32),
        pltpu.VMEM_SHARED((_N,), _F32),
        pltpu.VMEM_SHARED((_N,), _F32),
        pltpu.SemaphoreType.DMA,
        pltpu.SemaphoreType.DMA,
        pltpu.SemaphoreType.DMA,
        pltpu.SemaphoreType.DMA,
    ),
  )


_scatter_f = _make_scatter(_E)


def _nmlp_body(h_ref, g0_ref, g1_ref, wh_ref, wg_ref,
               bn1_ref, wn2_ref, bn2_ref,
               cx_ref, cy_ref, cz_ref, x0, x1, y0, y1, z0, z1,
               hout_ref, cox_ref, coy_ref, coz_ref):
    @pl.when(pl.program_id(0) == 0)
    def _coords():
        cox_ref[...] = cx_ref[...] + x0[...] + x1[...]
        coy_ref[...] = cy_ref[...] + y0[...] + y1[...]
        coz_ref[...] = cz_ref[...] + z0[...] + z1[...]

    g = g0_ref[...] + g1_ref[...]
    pre = (_dotT(h_ref[...], wh_ref[...]) + _dotT(g, wg_ref[...])
           + bn1_ref[...][None, :])
    hm = jnp.maximum(pre, 0.0)
    hout_ref[...] = _dotT(hm, wn2_ref[...]) + bn2_ref[...][None, :]


_NBLK = 1000
_nmlp = pl.pallas_call(
    _nmlp_body,
    grid=(_N // _NBLK,),
    in_specs=[
        pl.BlockSpec((_NBLK, _D), lambda i: (i, 0)),
        pl.BlockSpec((_NBLK, _D), lambda i: (i, 0)),
        pl.BlockSpec((_NBLK, _D), lambda i: (i + _N // _NBLK, 0)),
        pl.BlockSpec((_D, _D), lambda i: (0, 0)),
        pl.BlockSpec((_D, _D), lambda i: (0, 0)),
        pl.BlockSpec((_D,), lambda i: (0,)),
        pl.BlockSpec((_D, _D), lambda i: (0, 0)),
        pl.BlockSpec((_D,), lambda i: (0,)),
    ] + [pl.BlockSpec((_N,), lambda i: (0,))] * 9,
    out_specs=[pl.BlockSpec((_NBLK, _D), lambda i: (i, 0))]
    + [pl.BlockSpec((_N,), lambda i: (0,))] * 3,
    out_shape=[jax.ShapeDtypeStruct((_N, _D), _F32)]
    + [jax.ShapeDtypeStruct((_N,), _F32)] * 3,
)


@jax.jit
def kernel(h, coords, edge_index, We1, be1, We2, be2, Wn1, bn1, Wn2, bn2,
           Wc1, bc1, Wc2):
    row = edge_index[0]
    col = edge_index[1]
    wr = We1[:, :_D]
    wcl = We1[:, _D:2 * _D]
    w1d = We1[:, 2 * _D]
    cx = coords[:, 0]
    cy = coords[:, 1]
    cz = coords[:, 2]

    a, b = _pre(h, wr, wcl, be1)
    u, rx, ry, rz = _edge_gather_f(a, b, cx, cy, cz, row, col, w1d)
    m, cw = _emlp_f(u, We2, be2, Wc1, bc1, Wc2[0])
    (aggp, cux0, cux1, cuy0, cuy1, cuz0, cuz1) = _scatter_f(
        m, cw, rx, ry, rz, row)
    wh = Wn1[:, :_D]
    wg = Wn1[:, _D:]
    h_new, cox, coy, coz = _nmlp(h, aggp, aggp, wh, wg, bn1, Wn2, bn2,
                                 cx, cy, cz, cux0, cux1, cuy0, cuy1,
                                 cuz0, cuz1)
    return h_new, jnp.stack([cox, coy, coz], axis=1)


# speedup vs baseline: 1.0836x; 1.0831x over previous
import functools

import jax
import jax.numpy as jnp
from jax import lax
from jax.experimental import pallas as pl
from jax.experimental.pallas import tpu as pltpu
from jax.experimental.pallas import tpu_sc as plsc

_N = 10000
_E = 320000
_D = 128

_NW = 32
_CH = 128
_NCH = _E // _CH
_PERW = -(-_NCH // _NW)

_F32 = jnp.float32


def _dotT(x, w):
    return lax.dot_general(x, w, (((1,), (1,)), ((), ())),
                           preferred_element_type=_F32)


def _pre_body(h_ref, wr_ref, wc_ref, be1_ref, a_ref, b_ref):
    h = h_ref[...]
    a_ref[...] = _dotT(h, wr_ref[...]) + be1_ref[...][None, :]
    b_ref[...] = _dotT(h, wc_ref[...])


_pre = pl.pallas_call(
    _pre_body,
    grid=(10,),
    in_specs=[
        pl.BlockSpec((1000, _D), lambda i: (i, 0)),
        pl.BlockSpec((_D, _D), lambda i: (0, 0)),
        pl.BlockSpec((_D, _D), lambda i: (0, 0)),
        pl.BlockSpec((_D,), lambda i: (0,)),
    ],
    out_specs=[pl.BlockSpec((1000, _D), lambda i: (i, 0))] * 2,
    out_shape=[jax.ShapeDtypeStruct((_N, _D), _F32)] * 2,
)


def _make_edge_gather(ne):
  nch = ne // _CH
  perw = -(-nch // _NW)

  def _edge_gather_body(a_hbm, b_hbm, cx_hbm, cy_hbm, cz_hbm, row_hbm, col_hbm,
                      w1_hbm,
                      u_hbm, rx_hbm, ry_hbm, rz_hbm,
                      idxr, idxc, bufa, bufb, crx, cry, crz, ccx, ccy, ccz,
                      brx, bry, brz, sqbuf0, sqbuf1, w1vm,
                      semi0, semi1, semg0, semg1, semw0, semw1):
      cid = lax.axis_index("c")
      sid = lax.axis_index("s")
      wid = sid * 2 + cid
      pltpu.sync_copy(w1_hbm, w1vm)
      w1c = [w1vm[pl.ds(i * 16, 16)] for i in range(8)]
      nloc = jnp.minimum(perw, nch - wid * perw)
      semi = (semi0, semi1)
      semg = (semg0, semg1)
      semw = (semw0, semw1)

      def base_of(k):
          return (wid * perw + k) * _CH

      def fire_idx(k, p):
          b = base_of(k)
          pltpu.async_copy(row_hbm.at[pl.ds(b, _CH)], idxr.at[p], semi[p])
          pltpu.async_copy(col_hbm.at[pl.ds(b, _CH)], idxc.at[p], semi[p])

      def wait_idx(p):
          pltpu.make_async_copy(row_hbm.at[pl.ds(0, _CH)], idxr.at[p],
                                semi[p]).wait()
          pltpu.make_async_copy(col_hbm.at[pl.ds(0, _CH)], idxc.at[p],
                                semi[p]).wait()

      def gather_copies(p):
          s = semg[p]
          return (
              pltpu.make_async_copy(a_hbm.at[idxr.at[p]], bufa.at[p], s),
              pltpu.make_async_copy(b_hbm.at[idxc.at[p]], bufb.at[p], s),
              pltpu.make_async_copy(cx_hbm.at[idxr.at[p]], crx.at[p], s),
              pltpu.make_async_copy(cy_hbm.at[idxr.at[p]], cry.at[p], s),
              pltpu.make_async_copy(cz_hbm.at[idxr.at[p]], crz.at[p], s),
              pltpu.make_async_copy(cx_hbm.at[idxc.at[p]], ccx.at[p], s),
              pltpu.make_async_copy(cy_hbm.at[idxc.at[p]], ccy.at[p], s),
              pltpu.make_async_copy(cz_hbm.at[idxc.at[p]], ccz.at[p], s),
          )

      def fire_gather(p):
          for c in gather_copies(p):
              c.start()

      def wait_gather(p):
          for c in gather_copies(p):
              c.wait()

      def write_copies(k, p):
          b = base_of(k)
          s = semw[p]
          return (
              pltpu.make_async_copy(bufa.at[p], u_hbm.at[pl.ds(b, _CH)], s),
              pltpu.make_async_copy(brx.at[p], rx_hbm.at[pl.ds(b, _CH)], s),
              pltpu.make_async_copy(bry.at[p], ry_hbm.at[pl.ds(b, _CH)], s),
              pltpu.make_async_copy(brz.at[p], rz_hbm.at[pl.ds(b, _CH)], s),
          )

      def fire_writes(k, p):
          for c in write_copies(k, p):
              c.start()

      def wait_writes(p):
          for c in write_copies(0, p):
              c.wait()

      def compute(p):
          sq_b = (sqbuf0, sqbuf1)[p]
          for g in range(_CH // 16):
              sl = pl.ds(g * 16, 16)
              dx = crx[p, sl] - ccx[p, sl]
              dy = cry[p, sl] - ccy[p, sl]
              dz = crz[p, sl] - ccz[p, sl]
              brx[p, sl] = dx
              bry[p, sl] = dy
              brz[p, sl] = dz
              sq_b[sl] = dx * dx + dy * dy + dz * dz

          @pl.loop(0, _CH)
          def _edge(e):
              s = sq_b[pl.ds(e, 16)][0]
              for i in range(8):
                  sl = pl.ds(i * 16, 16)
                  z = bufa[p, e, sl] + bufb[p, e, sl] + s * w1c[i]
                  bufa[p, e, sl] = jnp.maximum(z, 0.0)

      fire_idx(0, 0)
      fire_idx(1, 1)
      wait_idx(0)
      fire_gather(0)

      @pl.loop(0, (nloc + 1) // 2)
      def _t(t):
          for p in (0, 1):
              k = 2 * t + p

              @pl.when(k < nloc)
              def _():
                  wait_gather(p)

                  @pl.when(k + 2 < nloc)
                  def _():
                      fire_idx(k + 2, p)

                  @pl.when(k + 1 < nloc)
                  def _():
                      wait_idx(1 - p)

                      @pl.when(k >= 1)
                      def _():
                          wait_writes(1 - p)

                      fire_gather(1 - p)

                  compute(p)
                  fire_writes(k, p)

      wait_writes(0)
      wait_writes(1)


  return pl.kernel(
    _edge_gather_body,
    out_type=(jax.ShapeDtypeStruct((ne, _D), _F32),
              jax.ShapeDtypeStruct((ne,), _F32),
              jax.ShapeDtypeStruct((ne,), _F32),
              jax.ShapeDtypeStruct((ne,), _F32)),
    mesh=plsc.VectorSubcoreMesh(core_axis_name="c", subcore_axis_name="s",
                                num_cores=2, num_subcores=16),
    scratch_types=(
        pltpu.VMEM((2, _CH), jnp.int32),
        pltpu.VMEM((2, _CH), jnp.int32),
        pltpu.VMEM((2, _CH, _D), _F32),
        pltpu.VMEM((2, _CH, _D), _F32),
        pltpu.VMEM((2, _CH), _F32),
        pltpu.VMEM((2, _CH), _F32),
        pltpu.VMEM((2, _CH), _F32),
        pltpu.VMEM((2, _CH), _F32),
        pltpu.VMEM((2, _CH), _F32),
        pltpu.VMEM((2, _CH), _F32),
        pltpu.VMEM((2, _CH), _F32),
        pltpu.VMEM((2, _CH), _F32),
        pltpu.VMEM((2, _CH), _F32),
        pltpu.VMEM((_CH + 16,), _F32),
        pltpu.VMEM((_CH + 16,), _F32),
        pltpu.VMEM((_D,), _F32),
        pltpu.SemaphoreType.DMA,
        pltpu.SemaphoreType.DMA,
        pltpu.SemaphoreType.DMA,
        pltpu.SemaphoreType.DMA,
        pltpu.SemaphoreType.DMA,
        pltpu.SemaphoreType.DMA,
    ),
  )


_edge_gather_f = _make_edge_gather(_E)


def _emlp_body(u_ref, we2_ref, be2_ref, wc1_ref, bc1_ref, wc2_ref,
               m_ref, cw_ref):
    u = u_ref[...]
    m = _dotT(u, we2_ref[...]) + be2_ref[...][None, :]
    m_ref[...] = m
    t = jnp.maximum(_dotT(m, wc1_ref[...]) + bc1_ref[...][None, :], 0.0)
    cw_ref[...] = jnp.sum(t * wc2_ref[...][None, :], axis=1)


_EBLK = 512
_emlp_f = pl.pallas_call(
    _emlp_body,
    grid=(_E // _EBLK,),
    in_specs=[
        pl.BlockSpec((_EBLK, _D), lambda i: (i, 0)),
        pl.BlockSpec((_D, _D), lambda i: (0, 0)),
        pl.BlockSpec((_D,), lambda i: (0,)),
        pl.BlockSpec((_D, _D), lambda i: (0, 0)),
        pl.BlockSpec((_D,), lambda i: (0,)),
        pl.BlockSpec((_D,), lambda i: (0,)),
    ],
    out_specs=[
        pl.BlockSpec((_EBLK, _D), lambda i: (i, 0)),
        pl.BlockSpec((_EBLK,), lambda i: (i,)),
    ],
    out_shape=[
        jax.ShapeDtypeStruct((_E, _D), _F32),
        jax.ShapeDtypeStruct((_E,), _F32),
    ],
)


_NPT = 1000


def _make_scatter(ne):
  nch = ne // _CH
  perw = -(-nch // _NW)

  def _scatter_body(m_hbm, cw_hbm, rx_hbm, ry_hbm, rz_hbm, row_hbm,
                  aggp_hbm, cux0_hbm, cux1_hbm, cuy0_hbm, cuy1_hbm,
                  cuz0_hbm, cuz1_hbm,
                  idx, mbuf, cwbuf, brx, bry, brz, wx, wy, wz, ob1, zbm,
                  accm, accx, accy, accz,
                  seml0, seml1, sems0, sems1):
      cid = lax.axis_index("c")
      sid = lax.axis_index("s")
      wid = sid * 2 + cid

      @pl.loop(0, 40)
      def _zm(r):
          for i in range(8):
              zbm[r, pl.ds(i * 16, 16)] = jnp.zeros((16,), _F32)

      @pl.loop(0, 63)
      def _zo(r):
          ob1[pl.ds(r * 16, 16)] = jnp.zeros((16,), _F32)

      @pl.when(sid < 10)
      def _zero():
          for c in range(25):
              pltpu.sync_copy(
                  zbm,
                  accm.at[pl.ds(sid * _NPT + c * 40, 40)])
          sl = pl.ds(sid * _NPT, _NPT)
          pltpu.sync_copy(ob1.at[pl.ds(0, _NPT)], accx.at[sl])
          pltpu.sync_copy(ob1.at[pl.ds(0, _NPT)], accy.at[sl])
          pltpu.sync_copy(ob1.at[pl.ds(0, _NPT)], accz.at[sl])

      plsc.subcore_barrier()

      nloc = jnp.minimum(perw, nch - wid * perw)
      seml = (seml0, seml1)
      sems = (sems0, sems1)

      def base_of(k):
          return (wid * perw + k) * _CH

      def load_copies(k, p):
          b = base_of(k)
          s = seml[p]
          return (
              pltpu.make_async_copy(row_hbm.at[pl.ds(b, _CH)], idx.at[p], s),
              pltpu.make_async_copy(m_hbm.at[pl.ds(b, _CH)], mbuf.at[p], s),
              pltpu.make_async_copy(cw_hbm.at[pl.ds(b, _CH)], cwbuf.at[p], s),
              pltpu.make_async_copy(rx_hbm.at[pl.ds(b, _CH)], brx.at[p], s),
              pltpu.make_async_copy(ry_hbm.at[pl.ds(b, _CH)], bry.at[p], s),
              pltpu.make_async_copy(rz_hbm.at[pl.ds(b, _CH)], brz.at[p], s),
          )

      def fire_loads(k, p):
          for c in load_copies(k, p):
              c.start()

      def wait_loads(p):
          for c in load_copies(0, p):
              c.wait()

      def scat_copies(p):
          s = sems[p]
          return (
              pltpu.make_async_copy(mbuf.at[p], accm.at[idx.at[p]], s),
              pltpu.make_async_copy(wx.at[p], accx.at[idx.at[p]], s),
              pltpu.make_async_copy(wy.at[p], accy.at[idx.at[p]], s),
              pltpu.make_async_copy(wz.at[p], accz.at[idx.at[p]], s),
          )

      def fire_scat(p):
          for c in scat_copies(p):
              c.start(add=True)

      def wait_scat(p):
          for c in scat_copies(p):
              c.wait()

      fire_loads(0, 0)

      @pl.loop(0, (nloc + 1) // 2)
      def _t(t):
          for p in (0, 1):
              k = 2 * t + p

              @pl.when(k < nloc)
              def _():
                  wait_loads(p)

                  @pl.when(k + 1 < nloc)
                  def _():
                      @pl.when(k >= 1)
                      def _():
                          wait_scat(1 - p)

                      fire_loads(k + 1, 1 - p)

                  for g in range(_CH // 16):
                      sl = pl.ds(g * 16, 16)
                      cw16 = cwbuf[p, sl]
                      wx[p, sl] = cw16 * brx[p, sl]
                      wy[p, sl] = cw16 * bry[p, sl]
                      wz[p, sl] = cw16 * brz[p, sl]
                  fire_scat(p)

      wait_scat(0)
      wait_scat(1)

      plsc.subcore_barrier()

      @pl.when(sid < 10)
      def _out():
          for c in range(25):
              pltpu.sync_copy(
                  accm.at[pl.ds(sid * _NPT + c * 40, 40)],
                  zbm)
              pltpu.sync_copy(
                  zbm,
                  aggp_hbm.at[pl.ds(cid * _N + sid * _NPT + c * 40, 40)])
          sl = pl.ds(sid * _NPT, _NPT)
          for acc, o0_hbm, o1_hbm in ((accx, cux0_hbm, cux1_hbm),
                                      (accy, cuy0_hbm, cuy1_hbm),
                                      (accz, cuz0_hbm, cuz1_hbm)):
              pltpu.sync_copy(acc.at[sl], ob1.at[pl.ds(0, _NPT)])

              @pl.when(cid == 0)
              def _o0():
                  pltpu.sync_copy(ob1.at[pl.ds(0, _NPT)], o0_hbm.at[sl])

              @pl.when(cid == 1)
              def _o1():
                  pltpu.sync_copy(ob1.at[pl.ds(0, _NPT)], o1_hbm.at[sl])


  return pl.kernel(
    _scatter_body,
    out_type=(jax.ShapeDtypeStruct((2 * _N, _D), _F32),)
             + (jax.ShapeDtypeStruct((_N,), _F32),) * 6,
    mesh=plsc.VectorSubcoreMesh(core_axis_name="c", subcore_axis_name="s",
                                num_cores=2, num_subcores=16),
    scratch_types=(
        pltpu.VMEM((2, _CH), jnp.int32),
        pltpu.VMEM((2, _CH, _D), _F32),
        pltpu.VMEM((2, _CH), _F32),
        pltpu.VMEM((2, _CH), _F32),
        pltpu.VMEM((2, _CH), _F32),
        pltpu.VMEM((2, _CH), _F32),
        pltpu.VMEM((2, _CH), _F32),
        pltpu.VMEM((2, _CH), _F32),
        pltpu.VMEM((2, _CH), _F32),
        pltpu.VMEM((1008,), _F32),
        pltpu.VMEM((40, _D), _F32),
        pltpu.VMEM_SHARED((_N, _D), _F32),
        pltpu.VMEM_SHARED((_N,), _F32),
        pltpu.VMEM_SHARED((_N,), _F32),
        pltpu.VMEM_SHARED((_N,), _F32),
        pltpu.SemaphoreType.DMA,
        pltpu.SemaphoreType.DMA,
        pltpu.SemaphoreType.DMA,
        pltpu.SemaphoreType.DMA,
    ),
  )


_scatter_f = _make_scatter(_E)


def _nmlp_body(h_ref, g0_ref, g1_ref, wh_ref, wg_ref,
               bn1_ref, wn2_ref, bn2_ref,
               cx_ref, cy_ref, cz_ref, x0, x1, y0, y1, z0, z1,
               hout_ref, cox_ref, coy_ref, coz_ref):
    @pl.when(pl.program_id(0) == 0)
    def _coords():
        cox_ref[...] = cx_ref[...] + x0[...] + x1[...]
        coy_ref[...] = cy_ref[...] + y0[...] + y1[...]
        coz_ref[...] = cz_ref[...] + z0[...] + z1[...]

    g = g0_ref[...] + g1_ref[...]
    pre = (_dotT(h_ref[...], wh_ref[...]) + _dotT(g, wg_ref[...])
           + bn1_ref[...][None, :])
    hm = jnp.maximum(pre, 0.0)
    hout_ref[...] = _dotT(hm, wn2_ref[...]) + bn2_ref[...][None, :]


_NBLK = 1000
_nmlp = pl.pallas_call(
    _nmlp_body,
    grid=(_N // _NBLK,),
    in_specs=[
        pl.BlockSpec((_NBLK, _D), lambda i: (i, 0)),
        pl.BlockSpec((_NBLK, _D), lambda i: (i, 0)),
        pl.BlockSpec((_NBLK, _D), lambda i: (i + _N // _NBLK, 0)),
        pl.BlockSpec((_D, _D), lambda i: (0, 0)),
        pl.BlockSpec((_D, _D), lambda i: (0, 0)),
        pl.BlockSpec((_D,), lambda i: (0,)),
        pl.BlockSpec((_D, _D), lambda i: (0, 0)),
        pl.BlockSpec((_D,), lambda i: (0,)),
    ] + [pl.BlockSpec((_N,), lambda i: (0,))] * 9,
    out_specs=[pl.BlockSpec((_NBLK, _D), lambda i: (i, 0))]
    + [pl.BlockSpec((_N,), lambda i: (0,))] * 3,
    out_shape=[jax.ShapeDtypeStruct((_N, _D), _F32)]
    + [jax.ShapeDtypeStruct((_N,), _F32)] * 3,
)


@jax.jit
def kernel(h, coords, edge_index, We1, be1, We2, be2, Wn1, bn1, Wn2, bn2,
           Wc1, bc1, Wc2):
    row = edge_index[0]
    col = edge_index[1]
    wr = We1[:, :_D]
    wcl = We1[:, _D:2 * _D]
    w1d = We1[:, 2 * _D]
    cx = coords[:, 0]
    cy = coords[:, 1]
    cz = coords[:, 2]

    a, b = _pre(h, wr, wcl, be1)
    u, rx, ry, rz = _edge_gather_f(a, b, cx, cy, cz, row, col, w1d)
    m, cw = _emlp_f(u, We2, be2, Wc1, bc1, Wc2[0])
    (aggp, cux0, cux1, cuy0, cuy1, cuz0, cuz1) = _scatter_f(
        m, cw, rx, ry, rz, row)
    wh = Wn1[:, :_D]
    wg = Wn1[:, _D:]
    h_new, cox, coy, coz = _nmlp(h, aggp, aggp, wh, wg, bn1, Wn2, bn2,
                                 cx, cy, cz, cux0, cux1, cuy0, cuy1,
                                 cuz0, cuz1)
    return h_new, jnp.stack([cox, coy, coz], axis=1)
